# Initial kernel scaffold; baseline (speedup 1.0000x reference)
import jax
import jax.numpy as jnp
from jax.experimental import pallas as pl


def kernel(x, edge_index, batch, c0_W1, c0_b1, c0_W2, c0_b2, c1_W1, c1_b1, c1_W2, c1_b2, c2_W1, c2_b1, c2_W2, c2_b2, bn0_g, bn0_b, bn1_g, bn1_b, bn2_g, bn2_b, lin0_W, lin0_b):
    raise NotImplementedError("write your pallas kernel here")


# speedup vs baseline: 4.0553x; 4.0553x over previous
import functools

import jax
import jax.numpy as jnp
from jax import lax
from jax.experimental import pallas as pl
from jax.experimental.pallas import tpu as pltpu
import jax.experimental.pallas.tpu_sc as plsc

N = 10000
D = 128
G = 128
NC = 2
NS = 16
NW = NC * NS
CHUNK = 128
NPAD = 10112
ROWS_PER_TILE = NPAD // NS
DUMMY_ROW = N


def _make_sc_agg(kpt):
  mesh = plsc.VectorSubcoreMesh(core_axis_name="c", subcore_axis_name="s")

  @functools.partial(
      pl.kernel,
      out_type=jax.ShapeDtypeStruct((NC, NPAD, D), jnp.float32),
      mesh=mesh,
      scratch_types=[
          pltpu.VMEM((kpt, CHUNK), jnp.int32),
          pltpu.VMEM((kpt, CHUNK), jnp.int32),
          pltpu.VMEM((CHUNK, D), jnp.float32),
          pltpu.VMEM_SHARED((NPAD, D), jnp.float32),
          pltpu.SemaphoreType.DMA,
      ],
  )
  def sc_agg(h_hbm, srcs_hbm, dsts_hbm, zeros_hbm, out_hbm,
             src_v, dst_v, gbuf, acc_sh, sem):
    c = lax.axis_index("c")
    s = lax.axis_index("s")
    w = c * NS + s
    pltpu.sync_copy(zeros_hbm,
                    acc_sh.at[pl.ds(s * ROWS_PER_TILE, ROWS_PER_TILE)])
    pltpu.sync_copy(srcs_hbm.at[w], src_v)
    pltpu.sync_copy(dsts_hbm.at[w], dst_v)
    plsc.subcore_barrier()

    def body(k, carry):
      pltpu.async_copy(h_hbm.at[src_v.at[k]], gbuf, sem).wait()
      pltpu.sync_copy(gbuf, acc_sh.at[dst_v.at[k]], add=True)
      return carry

    lax.fori_loop(0, kpt, body, 0)
    plsc.subcore_barrier()
    pltpu.sync_copy(acc_sh.at[pl.ds(s * ROWS_PER_TILE, ROWS_PER_TILE)],
                    out_hbm.at[c, pl.ds(s * ROWS_PER_TILE, ROWS_PER_TILE)])

  return sc_agg


def _tc_layer_body(h_ref, part_ref, w1_ref, b1_ref, w2_ref, b2_ref,
                   g_ref, b_ref, o_ref):
  u = h_ref[...] + part_ref[0, :N, :] + part_ref[1, :N, :]
  a = jnp.maximum(
      jnp.dot(u, w1_ref[...], preferred_element_type=jnp.float32)
      + b1_ref[...], 0.0)
  v = jnp.dot(a, w2_ref[...], preferred_element_type=jnp.float32) + b2_ref[...]
  r = jnp.maximum(v, 0.0)
  mu = jnp.mean(r, axis=0, keepdims=True)
  var = jnp.mean(jnp.square(r - mu), axis=0, keepdims=True)
  o_ref[...] = g_ref[...] * (r - mu) * lax.rsqrt(var + 1e-5) + b_ref[...]


_tc_layer = pl.pallas_call(
    _tc_layer_body,
    out_shape=jax.ShapeDtypeStruct((N, D), jnp.float32),
)


def _pool_body(h_ref, batch_ref, w_ref, b_ref, o_ref):
  ids = batch_ref[...]
  oh = (ids == lax.broadcasted_iota(jnp.int32, (N, G), 1)).astype(jnp.float32)
  xpool = lax.dot_general(oh, h_ref[...],
                          dimension_numbers=(((0,), (0,)), ((), ())),
                          preferred_element_type=jnp.float32)
  o_ref[...] = jnp.dot(xpool, w_ref[...],
                       preferred_element_type=jnp.float32) + b_ref[...]


_pool = pl.pallas_call(
    _pool_body,
    out_shape=jax.ShapeDtypeStruct((G, 2 * D), jnp.float32),
)


@jax.jit
def kernel(x, edge_index, batch, c0_W1, c0_b1, c0_W2, c0_b2, c1_W1, c1_b1,
           c1_W2, c1_b2, c2_W1, c2_b1, c2_W2, c2_b2, bn0_g, bn0_b, bn1_g,
           bn1_b, bn2_g, bn2_b, lin0_W, lin0_b):
  src = edge_index[0].astype(jnp.int32)
  dst = edge_index[1].astype(jnp.int32)
  e = src.shape[0]
  kpt = -(-e // (NW * CHUNK))
  e_pad = kpt * NW * CHUNK
  src_p = jnp.concatenate(
      [src, jnp.zeros((e_pad - e,), jnp.int32)]).reshape(NW, kpt, CHUNK)
  dst_p = jnp.concatenate(
      [dst, jnp.full((e_pad - e,), DUMMY_ROW, jnp.int32)]).reshape(
          NW, kpt, CHUNK)
  zeros = jnp.zeros((ROWS_PER_TILE, D), jnp.float32)
  sc_agg = _make_sc_agg(kpt)

  layers = [
      (c0_W1, c0_b1, c0_W2, c0_b2, bn0_g, bn0_b),
      (c1_W1, c1_b1, c1_W2, c1_b2, bn1_g, bn1_b),
      (c2_W1, c2_b1, c2_W2, c2_b2, bn2_g, bn2_b),
  ]
  h = x
  for (w1, b1, w2, b2, g, b) in layers:
    part = sc_agg(h, src_p, dst_p, zeros)
    h = _tc_layer(h, part, w1, b1.reshape(1, D), w2, b2.reshape(1, D),
                  g.reshape(1, D), b.reshape(1, D))
  out = _pool(h, batch.astype(jnp.int32).reshape(N, 1), lin0_W,
              lin0_b.reshape(1, 2 * D))
  return (out, h)
